# baseline (speedup 1.0000x reference)
import functools

import jax
import jax.numpy as jnp
from jax import lax
from jax.experimental import pallas as pl
from jax.experimental.pallas import tpu as pltpu
from jax.experimental.pallas import tpu_sc as plsc

_IOU_THR = 0.5
_SCORE_THR = 0.05
_MAX_OUT = 100
_Q1 = 20
_QSTEP = 24
_L = 16
_KP = 128
_IMAX = 2**31 - 1


@functools.lru_cache(maxsize=None)
def _build(B, N, C):
    L = _L
    Npad = -(-N // L) * L
    NV = Npad // L
    KP = _KP
    neg = jnp.float32(-jnp.inf)

    mesh = plsc.VectorSubcoreMesh(core_axis_name="c", subcore_axis_name="s")
    ncores = mesh.num_cores
    nsub = mesh.num_subcores
    assert B <= ncores and C <= nsub and C <= L

    def body(scores_hbm, boxes_hbm, ob_hbm, os_hbm, oc_hbm,
             s_ref, y1_ref, x1_ref, y2_ref, x2_ref, ar_ref,
             outs_ref, outi_ref, meta_ref,
             mS_ref, mI_ref, mM_ref,
             selS_ref, selC_ref, selB_ref, selV_ref, selK_ref,
             ob_ref, os_ref, oc_ref,
             shS_ref, shI_ref, shM_ref):
        cid = lax.axis_index("c")
        sid = lax.axis_index("s")
        lanes = lax.iota(jnp.int32, L)
        lane0 = lanes == 0
        zero16 = jnp.zeros((L,), jnp.int32)
        one16 = jnp.full((L,), 1, jnp.int32)
        is_img = cid < B
        is_cls = jnp.logical_and(sid < C, is_img)

        @pl.when(is_cls)
        def _():
            pltpu.sync_copy(scores_hbm.at[cid, sid], s_ref)
            pltpu.sync_copy(boxes_hbm.at[cid, 0], y1_ref)
            pltpu.sync_copy(boxes_hbm.at[cid, 1], x1_ref)
            pltpu.sync_copy(boxes_hbm.at[cid, 2], y2_ref)
            pltpu.sync_copy(boxes_hbm.at[cid, 3], x2_ref)

        @pl.when(jnp.logical_not(is_cls))
        def _():
            def garbage_clear(i, _):
                s_ref[pl.ds(i * L, L)] = jnp.full((L,), neg)
                return 0
            lax.fori_loop(0, NV, garbage_clear, 0)

        def init_body(i, carry):
            bestv, besti = carry
            ds = pl.ds(i * L, L)
            y1 = y1_ref[ds]
            x1 = x1_ref[ds]
            y2 = y2_ref[ds]
            x2 = x2_ref[ds]
            ar_ref[ds] = (y2 - y1) * (x2 - x1)
            s = s_ref[ds]
            s = jnp.where(s > _SCORE_THR, s, neg)
            s_ref[ds] = s
            gt = s > bestv
            bestv = jnp.where(gt, s, bestv)
            besti = jnp.where(gt, jnp.full((L,), i, jnp.int32), besti)
            return bestv, besti

        bestv, besti = lax.fori_loop(
            0, NV, init_body,
            (jnp.full((L,), neg), jnp.zeros((L,), jnp.int32)), unroll=2)
        m0 = jnp.max(bestv)
        bi0 = jnp.min(jnp.where(bestv == m0, besti * L + lanes,
                                jnp.int32(_IMAX)))

        def zero_body(i, _):
            ds = pl.ds(i * L, L)
            outs_ref[ds] = jnp.full((L,), neg)
            outi_ref[ds] = jnp.zeros((L,), jnp.int32)
            return 0

        lax.fori_loop(0, KP // L, zero_body, 0)

        c24 = jnp.float32(2.0**-24)

        def produce_cond(carry):
            k, mA, _, _, _, _, target = carry
            return jnp.logical_and(k < target, mA > neg)

        def produce_body(carry):
            k, mA, biA, mB, biB, hasB, target = carry
            ks = jnp.full((L,), k, jnp.int32)
            plsc.store_scatter(outs_ref, [ks], jnp.full((L,), mA), mask=lane0)
            plsc.store_scatter(outi_ref, [ks],
                               jnp.full((L,), biA, jnp.int32), mask=lane0)
            hasBb = hasB > 0
            maskB = jnp.logical_and(lane0, hasBb)
            ks1 = jnp.full((L,), k + 1, jnp.int32)
            plsc.store_scatter(outs_ref, [ks1], jnp.full((L,), mB), mask=maskB)
            plsc.store_scatter(outi_ref, [ks1],
                               jnp.full((L,), biB, jnp.int32), mask=maskB)
            k = k + 1 + hasB
            biBe = jnp.where(hasBb, biB, biA)
            bisA = jnp.full((L,), biA, jnp.int32)
            bisB = jnp.full((L,), biBe, jnp.int32)
            ay1 = plsc.load_gather(y1_ref, [bisA])
            ax1 = plsc.load_gather(x1_ref, [bisA])
            ay2 = plsc.load_gather(y2_ref, [bisA])
            ax2 = plsc.load_gather(x2_ref, [bisA])
            aar = plsc.load_gather(ar_ref, [bisA])
            by1 = plsc.load_gather(y1_ref, [bisB])
            bx1 = plsc.load_gather(x1_ref, [bisB])
            by2 = plsc.load_gather(y2_ref, [bisB])
            bx2 = plsc.load_gather(x2_ref, [bisB])
            bar = plsc.load_gather(ar_ref, [bisB])
            plsc.store_scatter(s_ref, [bisA], jnp.full((L,), neg), mask=lane0)
            plsc.store_scatter(s_ref, [bisB], jnp.full((L,), neg), mask=lane0)

            def sup(i, carry2):
                b1, i1, b2, i2 = carry2
                ds = pl.ds(i * L, L)
                y1 = y1_ref[ds]
                x1 = x1_ref[ds]
                y2 = y2_ref[ds]
                x2 = x2_ref[ds]
                s = s_ref[ds]
                ar = ar_ref[ds]
                iA = (jnp.maximum(jnp.minimum(y2, ay2) - jnp.maximum(y1, ay1), 0.0)
                      * jnp.maximum(jnp.minimum(x2, ax2) - jnp.maximum(x1, ax1), 0.0))
                uA = (ar + aar) - iA
                supA = (2.0 * iA - uA) > uA * c24
                iB = (jnp.maximum(jnp.minimum(y2, by2) - jnp.maximum(y1, by1), 0.0)
                      * jnp.maximum(jnp.minimum(x2, bx2) - jnp.maximum(x1, bx1), 0.0))
                uB = (ar + bar) - iB
                supB = (2.0 * iB - uB) > uB * c24
                s = jnp.where(jnp.logical_or(supA, supB), neg, s)
                s_ref[ds] = s
                blk = jnp.full((L,), i, jnp.int32)
                gt1 = s > b1
                gt2 = s > b2
                b2 = jnp.where(gt1, b1, jnp.where(gt2, s, b2))
                i2 = jnp.where(gt1, i1, jnp.where(gt2, blk, i2))
                b1 = jnp.where(gt1, s, b1)
                i1 = jnp.where(gt1, blk, i1)
                return b1, i1, b2, i2

            b1v, i1v, b2v, i2v = lax.fori_loop(
                0, NV, sup,
                (jnp.full((L,), neg), jnp.zeros((L,), jnp.int32),
                 jnp.full((L,), neg), jnp.zeros((L,), jnp.int32)), unroll=8)
            M1 = jnp.max(b1v)
            f1 = i1v * L + lanes
            B1 = jnp.min(jnp.where(b1v == M1, f1, jnp.int32(_IMAX)))
            l1 = jnp.bitwise_and(B1, L - 1)
            b1p = jnp.where(lanes == l1, b2v, b1v)
            fp = jnp.where(lanes == l1, i2v * L + lanes, f1)
            M2 = jnp.max(b1p)
            B2 = jnp.min(jnp.where(b1p == M2, fp, jnp.int32(_IMAX)))
            bis1 = jnp.full((L,), B1, jnp.int32)
            bis2 = jnp.full((L,), B2, jnp.int32)
            p1y1 = plsc.load_gather(y1_ref, [bis1])
            p1x1 = plsc.load_gather(x1_ref, [bis1])
            p1y2 = plsc.load_gather(y2_ref, [bis1])
            p1x2 = plsc.load_gather(x2_ref, [bis1])
            p1ar = plsc.load_gather(ar_ref, [bis1])
            p2y1 = plsc.load_gather(y1_ref, [bis2])
            p2x1 = plsc.load_gather(x1_ref, [bis2])
            p2y2 = plsc.load_gather(y2_ref, [bis2])
            p2x2 = plsc.load_gather(x2_ref, [bis2])
            p2ar = plsc.load_gather(ar_ref, [bis2])
            i12 = (jnp.maximum(jnp.minimum(p1y2, p2y2) - jnp.maximum(p1y1, p2y1), 0.0)
                   * jnp.maximum(jnp.minimum(p1x2, p2x2) - jnp.maximum(p1x1, p2x1), 0.0))
            u12 = (p1ar + p2ar) - i12
            sup12 = (2.0 * i12 - u12) > u12 * c24
            sup12s = jnp.max(jnp.where(sup12, jnp.int32(1), jnp.int32(0)))
            hasB_new = jnp.where(
                jnp.logical_and(M2 > neg, sup12s == 0),
                jnp.int32(1), jnp.int32(0))
            return k, M1, B1, M2, B2, hasB_new, target

        def round_cond(carry):
            done = carry[7]
            rounds = carry[8]
            return jnp.logical_and(done == 0, rounds < 8)

        def round_body(carry):
            k, m, bi, mB, biB, hasB, target, done, rounds = carry
            k, m, bi, mB, biB, hasB, _ = lax.while_loop(
                produce_cond, produce_body, (k, m, bi, mB, biB, hasB, target))
            pltpu.sync_copy(outs_ref, shS_ref.at[cid, sid])
            pltpu.sync_copy(outi_ref, shI_ref.at[cid, sid])
            exh = jnp.where(m > neg, jnp.int32(0), jnp.int32(1))
            plsc.store_scatter(meta_ref, [zero16],
                               jnp.full((L,), k, jnp.int32), mask=lane0)
            plsc.store_scatter(meta_ref, [one16],
                               jnp.full((L,), exh, jnp.int32), mask=lane0)
            pltpu.sync_copy(meta_ref, shM_ref.at[cid, sid])
            plsc.subcore_barrier()
            pltpu.sync_copy(shS_ref.at[cid], mS_ref)
            pltpu.sync_copy(shM_ref.at[cid], mM_ref)
            plsc.subcore_barrier()
            prod = plsc.load_gather(mM_ref, [lanes, zero16])
            exhv = plsc.load_gather(mM_ref, [lanes, one16])

            def sim(_, ptrs):
                heads = plsc.load_gather(mS_ref, [lanes, ptrs])
                mm = jnp.max(heads)
                valid = mm > neg
                csel = jnp.min(jnp.where(heads == mm, lanes, jnp.int32(_IMAX)))
                csel = jnp.minimum(csel, L - 1)
                ptrs = jnp.where(
                    jnp.logical_and(lanes == csel, valid), ptrs + 1, ptrs)
                return ptrs

            ptrs = lax.fori_loop(0, _MAX_OUT, sim, zero16)
            need = jnp.logical_and(
                jnp.logical_and(ptrs == prod, exhv == 0),
                prod < _MAX_OUT)
            done2 = jnp.where(jnp.any(need), jnp.int32(0), jnp.int32(1))
            my_need = jnp.max(jnp.where(
                jnp.logical_and(lanes == sid, need), jnp.int32(1),
                jnp.int32(0)))
            new_target = jnp.where(
                my_need > 0,
                jnp.minimum(k + jnp.int32(_QSTEP), jnp.int32(_MAX_OUT)),
                jnp.int32(0))
            return k, m, bi, mB, biB, hasB, new_target, done2, rounds + 1

        lax.while_loop(
            round_cond, round_body,
            (jnp.int32(0), m0, bi0, m0, jnp.int32(0), jnp.int32(0),
             jnp.int32(_Q1), jnp.int32(0), jnp.int32(0)))

        @pl.when(jnp.logical_and(sid == 0, is_img))
        def _():
            pltpu.sync_copy(shI_ref.at[cid], mI_ref)

            def z1(i, _):
                ob_ref[pl.ds(i * L, L)] = jnp.zeros((L,), jnp.float32)
                return 0

            lax.fori_loop(0, (KP * 4) // L, z1, 0)

            def z2(i, _):
                ds = pl.ds(i * L, L)
                os_ref[ds] = jnp.zeros((L,), jnp.float32)
                oc_ref[ds] = jnp.zeros((L,), jnp.int32)
                selV_ref[ds] = jnp.zeros((L,), jnp.int32)
                selB_ref[ds] = jnp.zeros((L,), jnp.int32)
                return 0

            lax.fori_loop(0, KP // L, z2, 0)

            def sel_body(kk, ptrs):
                heads = plsc.load_gather(mS_ref, [lanes, ptrs])
                mm = jnp.max(heads)
                valid = mm > neg
                csel = jnp.min(jnp.where(heads == mm, lanes, jnp.int32(_IMAX)))
                csel = jnp.minimum(csel, L - 1)
                psel = jnp.max(jnp.where(lanes == csel, ptrs, 0))
                cs = jnp.full((L,), csel, jnp.int32)
                bk = jnp.max(plsc.load_gather(
                    mI_ref, [cs, jnp.full((L,), psel, jnp.int32)]))
                ks = jnp.full((L,), kk, jnp.int32)
                plsc.store_scatter(selS_ref, [ks], jnp.full((L,), mm),
                                   mask=lane0)
                plsc.store_scatter(selC_ref, [ks], cs, mask=lane0)
                plsc.store_scatter(selB_ref, [ks],
                                   jnp.full((L,), bk, jnp.int32), mask=lane0)
                vflag = jnp.where(valid, jnp.int32(1), jnp.int32(0))
                plsc.store_scatter(selV_ref, [ks],
                                   jnp.full((L,), vflag, jnp.int32),
                                   mask=lane0)
                ptrs = jnp.where(jnp.logical_and(lanes == csel, valid),
                                 ptrs + 1, ptrs)
                return ptrs

            lax.fori_loop(0, _MAX_OUT, sel_body, zero16)

            def dedup_body(kk, _):
                ks = jnp.full((L,), kk, jnp.int32)
                bk = jnp.max(plsc.load_gather(selB_ref, [ks]))
                vk = jnp.max(plsc.load_gather(selV_ref, [ks]))
                bks = jnp.full((L,), bk, jnp.int32)

                def scan_j(j, acc):
                    ds = pl.ds(j * L, L)
                    jidx = j * L + lanes
                    hit = jnp.logical_and(
                        jnp.logical_and(selB_ref[ds] == bks,
                                        selV_ref[ds] > 0),
                        jidx < kk)
                    return jnp.logical_or(acc, jnp.any(hit))

                dup = lax.fori_loop(0, KP // L, scan_j, jnp.bool_(False))
                keep = jnp.logical_and(vk > 0, jnp.logical_not(dup))
                kflag = jnp.where(keep, jnp.int32(1), jnp.int32(0))
                plsc.store_scatter(selK_ref, [ks],
                                   jnp.full((L,), kflag, jnp.int32),
                                   mask=lane0)
                return 0

            lax.fori_loop(0, _MAX_OUT, dedup_body, 0)

            def comp_body(kk, ptr):
                ks = jnp.full((L,), kk, jnp.int32)
                keep = jnp.max(plsc.load_gather(selK_ref, [ks]))
                keepb = keep > 0
                mval = jnp.max(plsc.load_gather(selS_ref, [ks]))
                cs = jnp.max(plsc.load_gather(selC_ref, [ks]))
                bk = jnp.max(plsc.load_gather(selB_ref, [ks]))
                ps = jnp.full((L,), ptr, jnp.int32)
                m0m = jnp.logical_and(lane0, keepb)
                plsc.store_scatter(os_ref, [ps], jnp.full((L,), mval),
                                   mask=m0m)
                plsc.store_scatter(oc_ref, [ps],
                                   jnp.full((L,), cs, jnp.int32), mask=m0m)
                bks = jnp.full((L,), bk, jnp.int32)
                cy1 = plsc.load_gather(y1_ref, [bks])
                cx1 = plsc.load_gather(x1_ref, [bks])
                cy2 = plsc.load_gather(y2_ref, [bks])
                cx2 = plsc.load_gather(x2_ref, [bks])
                cv = jnp.where(lanes == 0, cy1,
                               jnp.where(lanes == 1, cx1,
                                         jnp.where(lanes == 2, cy2, cx2)))
                plsc.store_scatter(ob_ref, [ptr * 4 + lanes], cv,
                                   mask=jnp.logical_and(lanes < 4, keepb))
                return ptr + jnp.where(keepb, jnp.int32(1), jnp.int32(0))

            lax.fori_loop(0, _MAX_OUT, comp_body, jnp.int32(0))
            pltpu.sync_copy(ob_ref, ob_hbm.at[cid])
            pltpu.sync_copy(os_ref, os_hbm.at[cid])
            pltpu.sync_copy(oc_ref, oc_hbm.at[cid])

    return pl.kernel(
        body,
        out_type=[jax.ShapeDtypeStruct((B, KP * 4), jnp.float32),
                  jax.ShapeDtypeStruct((B, KP), jnp.float32),
                  jax.ShapeDtypeStruct((B, KP), jnp.int32)],
        mesh=mesh,
        compiler_params=pltpu.CompilerParams(needs_layout_passes=False),
        scratch_types=[
            pltpu.VMEM((Npad,), jnp.float32),
            pltpu.VMEM((Npad,), jnp.float32),
            pltpu.VMEM((Npad,), jnp.float32),
            pltpu.VMEM((Npad,), jnp.float32),
            pltpu.VMEM((Npad,), jnp.float32),
            pltpu.VMEM((Npad,), jnp.float32),
            pltpu.VMEM((KP,), jnp.float32),
            pltpu.VMEM((KP,), jnp.int32),
            pltpu.VMEM((8,), jnp.int32),
            pltpu.VMEM((L, KP), jnp.float32),
            pltpu.VMEM((L, KP), jnp.int32),
            pltpu.VMEM((L, 8), jnp.int32),
            pltpu.VMEM((KP,), jnp.float32),
            pltpu.VMEM((KP,), jnp.int32),
            pltpu.VMEM((KP,), jnp.int32),
            pltpu.VMEM((KP,), jnp.int32),
            pltpu.VMEM((KP,), jnp.int32),
            pltpu.VMEM((KP * 4,), jnp.float32),
            pltpu.VMEM((KP,), jnp.float32),
            pltpu.VMEM((KP,), jnp.int32),
            pltpu.VMEM_SHARED((ncores, nsub, KP), jnp.float32),
            pltpu.VMEM_SHARED((ncores, nsub, KP), jnp.int32),
            pltpu.VMEM_SHARED((ncores, nsub, 8), jnp.int32),
        ],
    )


@jax.jit
def kernel(boxes, scores):
    B, N, C = scores.shape
    Npad = -(-N // _L) * _L
    scores_t = jnp.transpose(scores, (0, 2, 1)).astype(jnp.float32)
    boxes_t = jnp.transpose(boxes, (0, 2, 1)).astype(jnp.float32)
    if Npad != N:
        scores_t = jnp.pad(scores_t, ((0, 0), (0, 0), (0, Npad - N)))
        boxes_t = jnp.pad(boxes_t, ((0, 0), (0, 0), (0, Npad - N)))
    run = _build(B, N, C)
    obF, osF, ocF = run(scores_t, boxes_t)
    out_boxes = obF.reshape(B, _KP, 4)[:, :_MAX_OUT]
    out_scores = osF[:, :_MAX_OUT]
    out_classes = ocF[:, :_MAX_OUT]
    return out_boxes, out_scores, out_classes


# speedup vs baseline: 1.0329x; 1.0329x over previous
import functools

import jax
import jax.numpy as jnp
from jax import lax
from jax.experimental import pallas as pl
from jax.experimental.pallas import tpu as pltpu
from jax.experimental.pallas import tpu_sc as plsc

_IOU_THR = 0.5
_SCORE_THR = 0.05
_MAX_OUT = 100
_Q1 = 20
_QSTEP = 24
_L = 16
_KP = 128
_IMAX = 2**31 - 1


@functools.lru_cache(maxsize=None)
def _build(B, N, C):
    L = _L
    Npad = -(-N // L) * L
    NV = Npad // L
    KP = _KP
    neg = jnp.float32(-jnp.inf)

    mesh = plsc.VectorSubcoreMesh(core_axis_name="c", subcore_axis_name="s")
    ncores = mesh.num_cores
    nsub = mesh.num_subcores
    assert B <= ncores and C <= nsub and C <= L

    def body(scores_hbm, boxes_hbm, ob_hbm, os_hbm, oc_hbm,
             s_ref, y1_ref, x1_ref, y2_ref, x2_ref, ar_ref,
             outs_ref, outi_ref, meta_ref,
             mS_ref, mI_ref, mM_ref,
             selS_ref, selC_ref, selB_ref, selV_ref, selK_ref,
             ob_ref, os_ref, oc_ref,
             shS_ref, shI_ref, shM_ref):
        cid = lax.axis_index("c")
        sid = lax.axis_index("s")
        lanes = lax.iota(jnp.int32, L)
        lane0 = lanes == 0
        zero16 = jnp.zeros((L,), jnp.int32)
        one16 = jnp.full((L,), 1, jnp.int32)
        is_img = cid < B
        is_cls = jnp.logical_and(sid < C, is_img)

        @pl.when(is_cls)
        def _():
            pltpu.sync_copy(scores_hbm.at[cid, sid], s_ref)
            pltpu.sync_copy(boxes_hbm.at[cid, 0], y1_ref)
            pltpu.sync_copy(boxes_hbm.at[cid, 1], x1_ref)
            pltpu.sync_copy(boxes_hbm.at[cid, 2], y2_ref)
            pltpu.sync_copy(boxes_hbm.at[cid, 3], x2_ref)

        @pl.when(jnp.logical_not(is_cls))
        def _():
            def garbage_clear(i, _):
                s_ref[pl.ds(i * L, L)] = jnp.full((L,), neg)
                return 0
            lax.fori_loop(0, NV, garbage_clear, 0)

        def init_body(i, carry):
            bestv, besti = carry
            ds = pl.ds(i * L, L)
            y1 = y1_ref[ds]
            x1 = x1_ref[ds]
            y2 = y2_ref[ds]
            x2 = x2_ref[ds]
            ar_ref[ds] = (y2 - y1) * (x2 - x1)
            s = s_ref[ds]
            s = jnp.where(s > _SCORE_THR, s, neg)
            s_ref[ds] = s
            gt = s > bestv
            bestv = jnp.where(gt, s, bestv)
            besti = jnp.where(gt, jnp.full((L,), i, jnp.int32), besti)
            return bestv, besti

        bestv, besti = lax.fori_loop(
            0, NV, init_body,
            (jnp.full((L,), neg), jnp.zeros((L,), jnp.int32)), unroll=2)
        m0 = jnp.max(bestv)
        bi0 = jnp.min(jnp.where(bestv == m0, besti * L + lanes,
                                jnp.int32(_IMAX)))

        def zero_body(i, _):
            ds = pl.ds(i * L, L)
            outs_ref[ds] = jnp.full((L,), neg)
            outi_ref[ds] = jnp.zeros((L,), jnp.int32)
            return 0

        lax.fori_loop(0, KP // L, zero_body, 0)

        c24 = jnp.float32(2.0**-24)

        def produce_cond(carry):
            k, mA, _, _, _, _, target = carry
            return jnp.logical_and(k < target, mA > neg)

        def produce_body(carry):
            k, mA, biA, mB, biB, hasB, target = carry
            ks = jnp.full((L,), k, jnp.int32)
            plsc.store_scatter(outs_ref, [ks], jnp.full((L,), mA), mask=lane0)
            plsc.store_scatter(outi_ref, [ks],
                               jnp.full((L,), biA, jnp.int32), mask=lane0)
            hasBb = hasB > 0
            maskB = jnp.logical_and(lane0, hasBb)
            ks1 = jnp.full((L,), k + 1, jnp.int32)
            plsc.store_scatter(outs_ref, [ks1], jnp.full((L,), mB), mask=maskB)
            plsc.store_scatter(outi_ref, [ks1],
                               jnp.full((L,), biB, jnp.int32), mask=maskB)
            k = k + 1 + hasB
            biBe = jnp.where(hasBb, biB, biA)
            bisA = jnp.full((L,), biA, jnp.int32)
            bisB = jnp.full((L,), biBe, jnp.int32)
            ay1 = plsc.load_gather(y1_ref, [bisA])
            ax1 = plsc.load_gather(x1_ref, [bisA])
            ay2 = plsc.load_gather(y2_ref, [bisA])
            ax2 = plsc.load_gather(x2_ref, [bisA])
            aar = plsc.load_gather(ar_ref, [bisA])
            by1 = plsc.load_gather(y1_ref, [bisB])
            bx1 = plsc.load_gather(x1_ref, [bisB])
            by2 = plsc.load_gather(y2_ref, [bisB])
            bx2 = plsc.load_gather(x2_ref, [bisB])
            bar = plsc.load_gather(ar_ref, [bisB])
            plsc.store_scatter(s_ref, [bisA], jnp.full((L,), neg), mask=lane0)
            plsc.store_scatter(s_ref, [bisB], jnp.full((L,), neg), mask=lane0)

            def sup(i, carry2):
                b1, i1, b2, i2 = carry2
                ds = pl.ds(i * L, L)
                y1 = y1_ref[ds]
                x1 = x1_ref[ds]
                y2 = y2_ref[ds]
                x2 = x2_ref[ds]
                s = s_ref[ds]
                ar = ar_ref[ds]
                iA = (jnp.maximum(jnp.minimum(y2, ay2) - jnp.maximum(y1, ay1), 0.0)
                      * jnp.maximum(jnp.minimum(x2, ax2) - jnp.maximum(x1, ax1), 0.0))
                uA = (ar + aar) - iA
                supA = (2.0 * iA - uA) > uA * c24
                iB = (jnp.maximum(jnp.minimum(y2, by2) - jnp.maximum(y1, by1), 0.0)
                      * jnp.maximum(jnp.minimum(x2, bx2) - jnp.maximum(x1, bx1), 0.0))
                uB = (ar + bar) - iB
                supB = (2.0 * iB - uB) > uB * c24
                s = jnp.where(jnp.logical_or(supA, supB), neg, s)
                s_ref[ds] = s
                blk = jnp.full((L,), i, jnp.int32)
                gt1 = s > b1
                gt2 = s > b2
                b2 = jnp.where(gt1, b1, jnp.where(gt2, s, b2))
                i2 = jnp.where(gt1, i1, jnp.where(gt2, blk, i2))
                b1 = jnp.where(gt1, s, b1)
                i1 = jnp.where(gt1, blk, i1)
                return b1, i1, b2, i2

            b1v, i1v, b2v, i2v = lax.fori_loop(
                0, NV, sup,
                (jnp.full((L,), neg), jnp.zeros((L,), jnp.int32),
                 jnp.full((L,), neg), jnp.zeros((L,), jnp.int32)), unroll=2)
            M1 = jnp.max(b1v)
            f1 = i1v * L + lanes
            B1 = jnp.min(jnp.where(b1v == M1, f1, jnp.int32(_IMAX)))
            l1 = jnp.bitwise_and(B1, L - 1)
            b1p = jnp.where(lanes == l1, b2v, b1v)
            fp = jnp.where(lanes == l1, i2v * L + lanes, f1)
            M2 = jnp.max(b1p)
            B2 = jnp.min(jnp.where(b1p == M2, fp, jnp.int32(_IMAX)))
            bis1 = jnp.full((L,), B1, jnp.int32)
            bis2 = jnp.full((L,), B2, jnp.int32)
            p1y1 = plsc.load_gather(y1_ref, [bis1])
            p1x1 = plsc.load_gather(x1_ref, [bis1])
            p1y2 = plsc.load_gather(y2_ref, [bis1])
            p1x2 = plsc.load_gather(x2_ref, [bis1])
            p1ar = plsc.load_gather(ar_ref, [bis1])
            p2y1 = plsc.load_gather(y1_ref, [bis2])
            p2x1 = plsc.load_gather(x1_ref, [bis2])
            p2y2 = plsc.load_gather(y2_ref, [bis2])
            p2x2 = plsc.load_gather(x2_ref, [bis2])
            p2ar = plsc.load_gather(ar_ref, [bis2])
            i12 = (jnp.maximum(jnp.minimum(p1y2, p2y2) - jnp.maximum(p1y1, p2y1), 0.0)
                   * jnp.maximum(jnp.minimum(p1x2, p2x2) - jnp.maximum(p1x1, p2x1), 0.0))
            u12 = (p1ar + p2ar) - i12
            sup12 = (2.0 * i12 - u12) > u12 * c24
            sup12s = jnp.max(jnp.where(sup12, jnp.int32(1), jnp.int32(0)))
            hasB_new = jnp.where(
                jnp.logical_and(M2 > neg, sup12s == 0),
                jnp.int32(1), jnp.int32(0))
            return k, M1, B1, M2, B2, hasB_new, target

        def round_cond(carry):
            done = carry[7]
            rounds = carry[8]
            return jnp.logical_and(done == 0, rounds < 8)

        def round_body(carry):
            k, m, bi, mB, biB, hasB, target, done, rounds = carry
            k, m, bi, mB, biB, hasB, _ = lax.while_loop(
                produce_cond, produce_body, (k, m, bi, mB, biB, hasB, target))
            pltpu.sync_copy(outs_ref, shS_ref.at[cid, sid])
            pltpu.sync_copy(outi_ref, shI_ref.at[cid, sid])
            exh = jnp.where(m > neg, jnp.int32(0), jnp.int32(1))
            plsc.store_scatter(meta_ref, [zero16],
                               jnp.full((L,), k, jnp.int32), mask=lane0)
            plsc.store_scatter(meta_ref, [one16],
                               jnp.full((L,), exh, jnp.int32), mask=lane0)
            pltpu.sync_copy(meta_ref, shM_ref.at[cid, sid])
            plsc.subcore_barrier()
            pltpu.sync_copy(shS_ref.at[cid], mS_ref)
            pltpu.sync_copy(shM_ref.at[cid], mM_ref)
            plsc.subcore_barrier()
            prod = plsc.load_gather(mM_ref, [lanes, zero16])
            exhv = plsc.load_gather(mM_ref, [lanes, one16])

            def sim(_, ptrs):
                heads = plsc.load_gather(mS_ref, [lanes, ptrs])
                mm = jnp.max(heads)
                valid = mm > neg
                csel = jnp.min(jnp.where(heads == mm, lanes, jnp.int32(_IMAX)))
                csel = jnp.minimum(csel, L - 1)
                ptrs = jnp.where(
                    jnp.logical_and(lanes == csel, valid), ptrs + 1, ptrs)
                return ptrs

            ptrs = lax.fori_loop(0, _MAX_OUT, sim, zero16)
            need = jnp.logical_and(
                jnp.logical_and(ptrs == prod, exhv == 0),
                prod < _MAX_OUT)
            done2 = jnp.where(jnp.any(need), jnp.int32(0), jnp.int32(1))
            my_need = jnp.max(jnp.where(
                jnp.logical_and(lanes == sid, need), jnp.int32(1),
                jnp.int32(0)))
            new_target = jnp.where(
                my_need > 0,
                jnp.minimum(k + jnp.int32(_QSTEP), jnp.int32(_MAX_OUT)),
                jnp.int32(0))
            return k, m, bi, mB, biB, hasB, new_target, done2, rounds + 1

        lax.while_loop(
            round_cond, round_body,
            (jnp.int32(0), m0, bi0, m0, jnp.int32(0), jnp.int32(0),
             jnp.int32(_Q1), jnp.int32(0), jnp.int32(0)))

        @pl.when(jnp.logical_and(sid == 0, is_img))
        def _():
            pltpu.sync_copy(shI_ref.at[cid], mI_ref)

            def z1(i, _):
                ob_ref[pl.ds(i * L, L)] = jnp.zeros((L,), jnp.float32)
                return 0

            lax.fori_loop(0, (KP * 4) // L, z1, 0)

            def z2(i, _):
                ds = pl.ds(i * L, L)
                os_ref[ds] = jnp.zeros((L,), jnp.float32)
                oc_ref[ds] = jnp.zeros((L,), jnp.int32)
                selV_ref[ds] = jnp.zeros((L,), jnp.int32)
                selB_ref[ds] = jnp.zeros((L,), jnp.int32)
                return 0

            lax.fori_loop(0, KP // L, z2, 0)

            def sel_body(kk, ptrs):
                heads = plsc.load_gather(mS_ref, [lanes, ptrs])
                mm = jnp.max(heads)
                valid = mm > neg
                csel = jnp.min(jnp.where(heads == mm, lanes, jnp.int32(_IMAX)))
                csel = jnp.minimum(csel, L - 1)
                psel = jnp.max(jnp.where(lanes == csel, ptrs, 0))
                cs = jnp.full((L,), csel, jnp.int32)
                bk = jnp.max(plsc.load_gather(
                    mI_ref, [cs, jnp.full((L,), psel, jnp.int32)]))
                ks = jnp.full((L,), kk, jnp.int32)
                plsc.store_scatter(selS_ref, [ks], jnp.full((L,), mm),
                                   mask=lane0)
                plsc.store_scatter(selC_ref, [ks], cs, mask=lane0)
                plsc.store_scatter(selB_ref, [ks],
                                   jnp.full((L,), bk, jnp.int32), mask=lane0)
                vflag = jnp.where(valid, jnp.int32(1), jnp.int32(0))
                plsc.store_scatter(selV_ref, [ks],
                                   jnp.full((L,), vflag, jnp.int32),
                                   mask=lane0)
                ptrs = jnp.where(jnp.logical_and(lanes == csel, valid),
                                 ptrs + 1, ptrs)
                return ptrs

            lax.fori_loop(0, _MAX_OUT, sel_body, zero16)

            def dedup_body(kk, _):
                ks = jnp.full((L,), kk, jnp.int32)
                bk = jnp.max(plsc.load_gather(selB_ref, [ks]))
                vk = jnp.max(plsc.load_gather(selV_ref, [ks]))
                bks = jnp.full((L,), bk, jnp.int32)

                def scan_j(j, acc):
                    ds = pl.ds(j * L, L)
                    jidx = j * L + lanes
                    hit = jnp.logical_and(
                        jnp.logical_and(selB_ref[ds] == bks,
                                        selV_ref[ds] > 0),
                        jidx < kk)
                    return jnp.logical_or(acc, jnp.any(hit))

                dup = lax.fori_loop(0, KP // L, scan_j, jnp.bool_(False))
                keep = jnp.logical_and(vk > 0, jnp.logical_not(dup))
                kflag = jnp.where(keep, jnp.int32(1), jnp.int32(0))
                plsc.store_scatter(selK_ref, [ks],
                                   jnp.full((L,), kflag, jnp.int32),
                                   mask=lane0)
                return 0

            lax.fori_loop(0, _MAX_OUT, dedup_body, 0)

            def comp_body(kk, ptr):
                ks = jnp.full((L,), kk, jnp.int32)
                keep = jnp.max(plsc.load_gather(selK_ref, [ks]))
                keepb = keep > 0
                mval = jnp.max(plsc.load_gather(selS_ref, [ks]))
                cs = jnp.max(plsc.load_gather(selC_ref, [ks]))
                bk = jnp.max(plsc.load_gather(selB_ref, [ks]))
                ps = jnp.full((L,), ptr, jnp.int32)
                m0m = jnp.logical_and(lane0, keepb)
                plsc.store_scatter(os_ref, [ps], jnp.full((L,), mval),
                                   mask=m0m)
                plsc.store_scatter(oc_ref, [ps],
                                   jnp.full((L,), cs, jnp.int32), mask=m0m)
                bks = jnp.full((L,), bk, jnp.int32)
                cy1 = plsc.load_gather(y1_ref, [bks])
                cx1 = plsc.load_gather(x1_ref, [bks])
                cy2 = plsc.load_gather(y2_ref, [bks])
                cx2 = plsc.load_gather(x2_ref, [bks])
                cv = jnp.where(lanes == 0, cy1,
                               jnp.where(lanes == 1, cx1,
                                         jnp.where(lanes == 2, cy2, cx2)))
                plsc.store_scatter(ob_ref, [ptr * 4 + lanes], cv,
                                   mask=jnp.logical_and(lanes < 4, keepb))
                return ptr + jnp.where(keepb, jnp.int32(1), jnp.int32(0))

            lax.fori_loop(0, _MAX_OUT, comp_body, jnp.int32(0))
            pltpu.sync_copy(ob_ref, ob_hbm.at[cid])
            pltpu.sync_copy(os_ref, os_hbm.at[cid])
            pltpu.sync_copy(oc_ref, oc_hbm.at[cid])

    return pl.kernel(
        body,
        out_type=[jax.ShapeDtypeStruct((B, KP * 4), jnp.float32),
                  jax.ShapeDtypeStruct((B, KP), jnp.float32),
                  jax.ShapeDtypeStruct((B, KP), jnp.int32)],
        mesh=mesh,
        compiler_params=pltpu.CompilerParams(needs_layout_passes=False),
        scratch_types=[
            pltpu.VMEM((Npad,), jnp.float32),
            pltpu.VMEM((Npad,), jnp.float32),
            pltpu.VMEM((Npad,), jnp.float32),
            pltpu.VMEM((Npad,), jnp.float32),
            pltpu.VMEM((Npad,), jnp.float32),
            pltpu.VMEM((Npad,), jnp.float32),
            pltpu.VMEM((KP,), jnp.float32),
            pltpu.VMEM((KP,), jnp.int32),
            pltpu.VMEM((8,), jnp.int32),
            pltpu.VMEM((L, KP), jnp.float32),
            pltpu.VMEM((L, KP), jnp.int32),
            pltpu.VMEM((L, 8), jnp.int32),
            pltpu.VMEM((KP,), jnp.float32),
            pltpu.VMEM((KP,), jnp.int32),
            pltpu.VMEM((KP,), jnp.int32),
            pltpu.VMEM((KP,), jnp.int32),
            pltpu.VMEM((KP,), jnp.int32),
            pltpu.VMEM((KP * 4,), jnp.float32),
            pltpu.VMEM((KP,), jnp.float32),
            pltpu.VMEM((KP,), jnp.int32),
            pltpu.VMEM_SHARED((ncores, nsub, KP), jnp.float32),
            pltpu.VMEM_SHARED((ncores, nsub, KP), jnp.int32),
            pltpu.VMEM_SHARED((ncores, nsub, 8), jnp.int32),
        ],
    )


@jax.jit
def kernel(boxes, scores):
    B, N, C = scores.shape
    Npad = -(-N // _L) * _L
    scores_t = jnp.transpose(scores, (0, 2, 1)).astype(jnp.float32)
    boxes_t = jnp.transpose(boxes, (0, 2, 1)).astype(jnp.float32)
    if Npad != N:
        scores_t = jnp.pad(scores_t, ((0, 0), (0, 0), (0, Npad - N)))
        boxes_t = jnp.pad(boxes_t, ((0, 0), (0, 0), (0, Npad - N)))
    run = _build(B, N, C)
    obF, osF, ocF = run(scores_t, boxes_t)
    out_boxes = obF.reshape(B, _KP, 4)[:, :_MAX_OUT]
    out_scores = osF[:, :_MAX_OUT]
    out_classes = ocF[:, :_MAX_OUT]
    return out_boxes, out_scores, out_classes
